# baseline (speedup 1.0000x reference)
import functools

import jax
import jax.numpy as jnp
from jax import lax
from jax.experimental import pallas as pl
from jax.experimental.pallas import tpu as pltpu
from jax.experimental.pallas import tpu_sc as plsc

NC = 2
NS = 16
NW = NC * NS

CHUNK = 128


def _edge_lin_body(ea_ref, we_ref, be_ref, out_ref):
    e = (
        lax.dot_general(
            ea_ref[...], we_ref[...],
            dimension_numbers=(((0,), (0,)), ((), ())),
            preferred_element_type=jnp.float32,
        )
        + be_ref[...]
    ).astype(jnp.bfloat16)
    out_ref[...] = pltpu.bitcast(e, jnp.int32)


def _edge_linear(edge_attr_t, We, be):
    DE, E = edge_attr_t.shape
    D = We.shape[1]
    BE = 16000
    grid = (E // BE,)
    return pl.pallas_call(
        _edge_lin_body,
        grid=grid,
        in_specs=[
            pl.BlockSpec((DE, BE), lambda i: (0, i)),
            pl.BlockSpec((DE, D), lambda i: (0, 0)),
            pl.BlockSpec((1, D), lambda i: (0, 0)),
        ],
        out_specs=pl.BlockSpec((BE // 2, D), lambda i: (i, 0)),
        out_shape=jax.ShapeDtypeStruct((E // 2, D), jnp.int32),
    )(edge_attr_t, We, be.reshape(1, D))


def _make_sc_edge(N, E, D, eoff=0):
    G = E // CHUNK
    nchunk = G // NW
    LEFT = G - nchunk * NW
    npad = ((N + NS * 8 - 1) // (NS * 8)) * NS * 8
    rpt = npad // NS

    mesh = plsc.VectorSubcoreMesh(
        core_axis_name="c", subcore_axis_name="s", num_cores=NC, num_subcores=NS
    )

    @functools.partial(
        pl.kernel,
        out_type=jax.ShapeDtypeStruct((NC, npad, D), jnp.float32),
        mesh=mesh,
        compiler_params=pltpu.CompilerParams(use_tc_tiling_on_sc=True),
        scratch_types=[
            pltpu.VMEM((CHUNK,), jnp.int32),
            pltpu.VMEM((CHUNK,), jnp.int32),
            pltpu.VMEM((CHUNK,), jnp.int32),
            pltpu.VMEM((CHUNK,), jnp.int32),
            pltpu.VMEM((CHUNK, D), jnp.float32),
            pltpu.VMEM((CHUNK, D), jnp.float32),
            pltpu.VMEM((CHUNK // 2, D), jnp.int32),
            pltpu.VMEM((CHUNK // 2, D), jnp.int32),
            pltpu.VMEM_SHARED((npad, D), jnp.float32),
            pltpu.SemaphoreType.DMA,
            pltpu.SemaphoreType.DMA,
            pltpu.SemaphoreType.DMA,
            pltpu.SemaphoreType.DMA,
            pltpu.SemaphoreType.DMA,
            pltpu.SemaphoreType.DMA,
            pltpu.SemaphoreType.DMA,
            pltpu.SemaphoreType.DMA,
            pltpu.SemaphoreType.DMA,
            pltpu.SemaphoreType.DMA,
        ],
    )
    def sc_edge(adj_hbm, feat_hbm, e_hbm, out_hbm,
                srcI0, srcI1, dstI0, dstI1, x0, x1, e0, e1, agg_sh,
                sem_si0, sem_si1, sem_di0, sem_di1,
                sem_x0, sem_x1, sem_e0, sem_e1, sem_s0, sem_s1):
        cid = lax.axis_index("c")
        sid = lax.axis_index("s")
        wid = sid * NC + cid
        srcI = (srcI0, srcI1)
        dstI = (dstI0, dstI1)
        xb = (x0, x1)
        eb = (e0, e1)
        sem_si = (sem_si0, sem_si1)
        sem_di = (sem_di0, sem_di1)
        sem_x = (sem_x0, sem_x1)
        sem_e = (sem_e0, sem_e1)
        sem_s = (sem_s0, sem_s1)

        def zbody(i, _):
            r = i // (D // 16)
            col = (i % (D // 16)) * 16
            x0[r, pl.ds(col, 16)] = jnp.zeros((16,), jnp.float32)
            return 0
        lax.fori_loop(0, CHUNK * (D // 16), zbody, 0, unroll=8)
        zbase = sid * rpt
        for j in range(rpt // CHUNK):
            pltpu.sync_copy(x0, agg_sh.at[pl.ds(zbase + j * CHUNK, CHUNK)])
        zrem = rpt % CHUNK
        if zrem:
            pltpu.sync_copy(
                x0.at[pl.ds(0, zrem)],
                agg_sh.at[pl.ds(zbase + (rpt // CHUNK) * CHUNK, zrem)])
        plsc.subcore_barrier()

        def issue_src(c, p):
            pltpu.async_copy(
                adj_hbm.at[0, pl.ds(eoff + (wid + NW * c) * CHUNK, CHUNK)], srcI[p],
                sem_si[p])

        def wait_src(c, p):
            pltpu.make_async_copy(
                adj_hbm.at[0, pl.ds(eoff + (wid + NW * c) * CHUNK, CHUNK)], srcI[p],
                sem_si[p]).wait()

        def issue_dst(c, p):
            pltpu.async_copy(
                adj_hbm.at[1, pl.ds(eoff + (wid + NW * c) * CHUNK, CHUNK)], dstI[p],
                sem_di[p])

        def wait_dst(c, p):
            pltpu.make_async_copy(
                adj_hbm.at[1, pl.ds(eoff + (wid + NW * c) * CHUNK, CHUNK)], dstI[p],
                sem_di[p]).wait()

        def issue_in(c, p):
            pltpu.async_copy(feat_hbm.at[srcI[p]], xb[p], sem_x[p])
            pltpu.async_copy(
                e_hbm.at[pl.ds((wid + NW * c) * (CHUNK // 2), CHUNK // 2)],
                eb[p], sem_e[p])

        def wait_in(c, p):
            pltpu.make_async_copy(feat_hbm.at[srcI[p]], xb[p], sem_x[p]).wait()
            pltpu.make_async_copy(
                e_hbm.at[pl.ds((wid + NW * c) * (CHUNK // 2), CHUNK // 2)],
                eb[p], sem_e[p]
            ).wait()

        def compute(p):
            hi_mask = jnp.full((16,), -65536, jnp.int32)

            @plsc.parallel_loop(0, CHUNK // 2, unroll=2)
            def _(rp):
                r0 = 2 * rp
                for k in range(D // 16):
                    sl = pl.ds(k * 16, 16)
                    w = eb[p][rp, sl]
                    elo = jax.lax.bitcast_convert_type(
                        jnp.left_shift(w, 16), jnp.float32)
                    ehi = jax.lax.bitcast_convert_type(
                        jnp.bitwise_and(w, hi_mask), jnp.float32)
                    xb[p][r0, sl] = jnp.maximum(xb[p][r0, sl] + elo, 0.0)
                    xb[p][r0 + 1, sl] = jnp.maximum(
                        xb[p][r0 + 1, sl] + ehi, 0.0)

        def issue_scatter(p):
            pltpu.async_copy(xb[p], agg_sh.at[dstI[p]], sem_s[p], add=True)

        def wait_scatter(p):
            pltpu.make_async_copy(xb[p], agg_sh.at[dstI[p]], sem_s[p]).wait()

        def step(c, p):
            q = 1 - p

            @pl.when(c + 1 < nchunk)
            def _():
                wait_src(c + 1, q)

                @pl.when(c >= 1)
                def _():
                    wait_scatter(q)
                issue_dst(c + 1, q)
                issue_in(c + 1, q)
            wait_in(c, p)

            @pl.when(c + 2 < nchunk)
            def _():
                issue_src(c + 2, p)
            compute(p)
            wait_dst(c, p)
            issue_scatter(p)

        issue_src(0, 0)
        issue_src(1, 1)
        issue_dst(0, 0)
        wait_src(0, 0)
        issue_in(0, 0)

        def pair(i, _):
            c = 2 * i
            step(c, 0)
            step(c + 1, 1)
            return 0

        lax.fori_loop(0, nchunk // 2, pair, 0)
        if nchunk % 2:
            step(jnp.int32(nchunk - 1), 0)

        wait_scatter(0)
        wait_scatter(1)

        @pl.when(wid < LEFT)
        def _():
            g = NW * nchunk + wid
            pltpu.sync_copy(adj_hbm.at[0, pl.ds(eoff + g * CHUNK, CHUNK)], srcI[0])
            pltpu.sync_copy(adj_hbm.at[1, pl.ds(eoff + g * CHUNK, CHUNK)], dstI[0])
            pltpu.sync_copy(e_hbm.at[pl.ds(g * (CHUNK // 2), CHUNK // 2)],
                            eb[0])
            pltpu.async_copy(feat_hbm.at[srcI[0]], xb[0], sem_x[0]).wait()
            compute(0)
            pltpu.sync_copy(xb[0], agg_sh.at[dstI[0]], add=True)

        plsc.subcore_barrier()
        pltpu.sync_copy(
            agg_sh.at[pl.ds(sid * rpt, rpt)],
            out_hbm.at[cid, pl.ds(sid * rpt, rpt)],
        )

    return sc_edge


def _mlp_body(f_ref, a0_ref, a1_ref, a2_ref, a3_ref, eps_ref, w1_ref, b1_ref,
              w2_ref, b2_ref, wl_ref, bl_ref, out_ref):
    h = (f_ref[...] * (1.0 + eps_ref[0, 0])
         + (a0_ref[0] + a1_ref[0]) + (a2_ref[0] + a3_ref[0]))
    h = jnp.dot(h, w1_ref[...], preferred_element_type=jnp.float32) + b1_ref[...]
    h = jnp.where(h > 0, h, 0.01 * h)
    h = jnp.dot(h, w2_ref[...], preferred_element_type=jnp.float32) + b2_ref[...]
    h = jnp.dot(h, wl_ref[...], preferred_element_type=jnp.float32) + bl_ref[...]
    out_ref[...] = jnp.where(h > 0, h, 0.01 * h)


def _mlp(feature, agg1, agg2, eps, W1, b1, W2, b2, Wl, bl):
    N, D = feature.shape
    BN = 2000
    grid = (N // BN,)
    row = lambda i: (i, 0)
    full = lambda i: (0, 0)
    return pl.pallas_call(
        _mlp_body,
        grid=grid,
        in_specs=[
            pl.BlockSpec((BN, D), row),
            pl.BlockSpec((1, BN, D), lambda i: (0, i, 0)),
            pl.BlockSpec((1, BN, D), lambda i: (1, i, 0)),
            pl.BlockSpec((1, BN, D), lambda i: (0, i, 0)),
            pl.BlockSpec((1, BN, D), lambda i: (1, i, 0)),
            pl.BlockSpec(memory_space=pltpu.SMEM),
            pl.BlockSpec((D, D), full),
            pl.BlockSpec((1, D), full),
            pl.BlockSpec((D, D), full),
            pl.BlockSpec((1, D), full),
            pl.BlockSpec((D, D), full),
            pl.BlockSpec((1, D), full),
        ],
        out_specs=pl.BlockSpec((BN, D), row),
        out_shape=jax.ShapeDtypeStruct((N, D), jnp.float32),
    )(feature, agg1, agg1, agg2, agg2, eps.reshape(1, 1), W1, b1.reshape(1, D),
      W2, b2.reshape(1, D), Wl, bl.reshape(1, D))


def kernel(feature, adj, edge_attr, We, be, eps, W1, b1, W2, b2, Wl, bl):
    N, D = feature.shape
    E = edge_attr.shape[0]
    adj32 = adj.astype(jnp.int32)
    Eh = E // 2
    eat = edge_attr.T

    e1 = _edge_linear(eat[:, :Eh], We, be)
    agg1 = _make_sc_edge(N, Eh, D, eoff=0)(adj32, feature, e1)
    e2 = _edge_linear(eat[:, Eh:], We, be)
    agg2 = _make_sc_edge(N, Eh, D, eoff=Eh)(adj32, feature, e2)
    return _mlp(feature, agg1, agg2, eps, W1, b1, W2, b2, Wl, bl)


# speedup vs baseline: 1.0776x; 1.0776x over previous
import functools

import jax
import jax.numpy as jnp
from jax import lax
from jax.experimental import pallas as pl
from jax.experimental.pallas import tpu as pltpu
from jax.experimental.pallas import tpu_sc as plsc

NC = 2
NS = 16
NW = NC * NS

CHUNK = 128


def _edge_lin_body(ea_ref, we_ref, be_ref, out_ref):
    e = (
        lax.dot_general(
            ea_ref[...], we_ref[...],
            dimension_numbers=(((0,), (0,)), ((), ())),
            preferred_element_type=jnp.float32,
        )
        + be_ref[...]
    ).astype(jnp.bfloat16)
    out_ref[...] = pltpu.bitcast(e, jnp.int32)


def _edge_linear(edge_attr_t, We, be):
    DE, E = edge_attr_t.shape
    D = We.shape[1]
    BE = 12800
    grid = (E // BE,)
    return pl.pallas_call(
        _edge_lin_body,
        grid=grid,
        in_specs=[
            pl.BlockSpec((DE, BE), lambda i: (0, i)),
            pl.BlockSpec((DE, D), lambda i: (0, 0)),
            pl.BlockSpec((1, D), lambda i: (0, 0)),
        ],
        out_specs=pl.BlockSpec((BE // 2, D), lambda i: (i, 0)),
        out_shape=jax.ShapeDtypeStruct((E // 2, D), jnp.int32),
    )(edge_attr_t, We, be.reshape(1, D))


def _make_sc_edge(N, E, D):
    G = E // CHUNK
    nchunk = G // NW
    LEFT = G - nchunk * NW
    npad = ((N + NS * 8 - 1) // (NS * 8)) * NS * 8
    rpt = npad // NS

    mesh = plsc.VectorSubcoreMesh(
        core_axis_name="c", subcore_axis_name="s", num_cores=NC, num_subcores=NS
    )

    @functools.partial(
        pl.kernel,
        out_type=jax.ShapeDtypeStruct((NC, npad, D), jnp.float32),
        mesh=mesh,
        compiler_params=pltpu.CompilerParams(use_tc_tiling_on_sc=True),
        scratch_types=[
            pltpu.VMEM((CHUNK,), jnp.int32),
            pltpu.VMEM((CHUNK,), jnp.int32),
            pltpu.VMEM((CHUNK,), jnp.int32),
            pltpu.VMEM((CHUNK,), jnp.int32),
            pltpu.VMEM((CHUNK, D), jnp.float32),
            pltpu.VMEM((CHUNK, D), jnp.float32),
            pltpu.VMEM((CHUNK // 2, D), jnp.int32),
            pltpu.VMEM((CHUNK // 2, D), jnp.int32),
            pltpu.VMEM_SHARED((npad, D), jnp.float32),
            pltpu.SemaphoreType.DMA,
            pltpu.SemaphoreType.DMA,
            pltpu.SemaphoreType.DMA,
            pltpu.SemaphoreType.DMA,
            pltpu.SemaphoreType.DMA,
            pltpu.SemaphoreType.DMA,
            pltpu.SemaphoreType.DMA,
            pltpu.SemaphoreType.DMA,
            pltpu.SemaphoreType.DMA,
            pltpu.SemaphoreType.DMA,
        ],
    )
    def sc_edge(adj_hbm, feat_hbm, e_hbm, out_hbm,
                srcI0, srcI1, dstI0, dstI1, x0, x1, e0, e1, agg_sh,
                sem_si0, sem_si1, sem_di0, sem_di1,
                sem_x0, sem_x1, sem_e0, sem_e1, sem_s0, sem_s1):
        cid = lax.axis_index("c")
        sid = lax.axis_index("s")
        wid = sid * NC + cid
        srcI = (srcI0, srcI1)
        dstI = (dstI0, dstI1)
        xb = (x0, x1)
        eb = (e0, e1)
        sem_si = (sem_si0, sem_si1)
        sem_di = (sem_di0, sem_di1)
        sem_x = (sem_x0, sem_x1)
        sem_e = (sem_e0, sem_e1)
        sem_s = (sem_s0, sem_s1)

        def zbody(i, _):
            r = i // (D // 16)
            col = (i % (D // 16)) * 16
            x0[r, pl.ds(col, 16)] = jnp.zeros((16,), jnp.float32)
            return 0
        lax.fori_loop(0, CHUNK * (D // 16), zbody, 0, unroll=8)
        zbase = sid * rpt
        for j in range(rpt // CHUNK):
            pltpu.sync_copy(x0, agg_sh.at[pl.ds(zbase + j * CHUNK, CHUNK)])
        zrem = rpt % CHUNK
        if zrem:
            pltpu.sync_copy(
                x0.at[pl.ds(0, zrem)],
                agg_sh.at[pl.ds(zbase + (rpt // CHUNK) * CHUNK, zrem)])
        plsc.subcore_barrier()

        def issue_src(c, p):
            pltpu.async_copy(
                adj_hbm.at[0, pl.ds((wid + NW * c) * CHUNK, CHUNK)], srcI[p],
                sem_si[p])

        def wait_src(c, p):
            pltpu.make_async_copy(
                adj_hbm.at[0, pl.ds((wid + NW * c) * CHUNK, CHUNK)], srcI[p],
                sem_si[p]).wait()

        def issue_dst(c, p):
            pltpu.async_copy(
                adj_hbm.at[1, pl.ds((wid + NW * c) * CHUNK, CHUNK)], dstI[p],
                sem_di[p])

        def wait_dst(c, p):
            pltpu.make_async_copy(
                adj_hbm.at[1, pl.ds((wid + NW * c) * CHUNK, CHUNK)], dstI[p],
                sem_di[p]).wait()

        def issue_in(c, p):
            pltpu.async_copy(feat_hbm.at[srcI[p]], xb[p], sem_x[p])
            pltpu.async_copy(
                e_hbm.at[pl.ds((wid + NW * c) * (CHUNK // 2), CHUNK // 2)],
                eb[p], sem_e[p])

        def wait_in(c, p):
            pltpu.make_async_copy(feat_hbm.at[srcI[p]], xb[p], sem_x[p]).wait()
            pltpu.make_async_copy(
                e_hbm.at[pl.ds((wid + NW * c) * (CHUNK // 2), CHUNK // 2)],
                eb[p], sem_e[p]
            ).wait()

        def compute(p):
            hi_mask = jnp.full((16,), -65536, jnp.int32)

            @plsc.parallel_loop(0, CHUNK // 2, unroll=2)
            def _(rp):
                r0 = 2 * rp
                for k in range(D // 16):
                    sl = pl.ds(k * 16, 16)
                    w = eb[p][rp, sl]
                    elo = jax.lax.bitcast_convert_type(
                        jnp.left_shift(w, 16), jnp.float32)
                    ehi = jax.lax.bitcast_convert_type(
                        jnp.bitwise_and(w, hi_mask), jnp.float32)
                    xb[p][r0, sl] = jnp.maximum(xb[p][r0, sl] + elo, 0.0)
                    xb[p][r0 + 1, sl] = jnp.maximum(
                        xb[p][r0 + 1, sl] + ehi, 0.0)

        def issue_scatter(p):
            pltpu.async_copy(xb[p], agg_sh.at[dstI[p]], sem_s[p], add=True)

        def wait_scatter(p):
            pltpu.make_async_copy(xb[p], agg_sh.at[dstI[p]], sem_s[p]).wait()

        def step(c, p):
            q = 1 - p

            @pl.when(c + 1 < nchunk)
            def _():
                wait_src(c + 1, q)

                @pl.when(c >= 1)
                def _():
                    wait_scatter(q)
                issue_dst(c + 1, q)
                issue_in(c + 1, q)
            wait_in(c, p)

            @pl.when(c + 2 < nchunk)
            def _():
                issue_src(c + 2, p)
            compute(p)
            wait_dst(c, p)
            issue_scatter(p)

        issue_src(0, 0)
        issue_src(1, 1)
        issue_dst(0, 0)
        wait_src(0, 0)
        issue_in(0, 0)

        def pair(i, _):
            c = 2 * i
            step(c, 0)
            step(c + 1, 1)
            return 0

        lax.fori_loop(0, nchunk // 2, pair, 0)

        wait_scatter(0)
        wait_scatter(1)

        @pl.when(wid < LEFT)
        def _():
            g = NW * nchunk + wid
            pltpu.sync_copy(adj_hbm.at[0, pl.ds(g * CHUNK, CHUNK)], srcI[0])
            pltpu.sync_copy(adj_hbm.at[1, pl.ds(g * CHUNK, CHUNK)], dstI[0])
            pltpu.sync_copy(e_hbm.at[pl.ds(g * (CHUNK // 2), CHUNK // 2)],
                            eb[0])
            pltpu.async_copy(feat_hbm.at[srcI[0]], xb[0], sem_x[0]).wait()
            compute(0)
            pltpu.sync_copy(xb[0], agg_sh.at[dstI[0]], add=True)

        plsc.subcore_barrier()
        pltpu.sync_copy(
            agg_sh.at[pl.ds(sid * rpt, rpt)],
            out_hbm.at[cid, pl.ds(sid * rpt, rpt)],
        )

    return sc_edge


def _mlp_body(f_ref, a0_ref, a1_ref, eps_ref, w1_ref, b1_ref, w2_ref, b2_ref,
              wl_ref, bl_ref, out_ref):
    h = (f_ref[...] * (1.0 + eps_ref[0, 0]) + a0_ref[0] + a1_ref[0])
    h = jnp.dot(h, w1_ref[...], preferred_element_type=jnp.float32) + b1_ref[...]
    h = jnp.where(h > 0, h, 0.01 * h)
    h = jnp.dot(h, w2_ref[...], preferred_element_type=jnp.float32) + b2_ref[...]
    h = jnp.dot(h, wl_ref[...], preferred_element_type=jnp.float32) + bl_ref[...]
    out_ref[...] = jnp.where(h > 0, h, 0.01 * h)


def _mlp(feature, agg, eps, W1, b1, W2, b2, Wl, bl):
    N, D = feature.shape
    BN = 2000
    grid = (N // BN,)
    row = lambda i: (i, 0)
    full = lambda i: (0, 0)
    return pl.pallas_call(
        _mlp_body,
        grid=grid,
        in_specs=[
            pl.BlockSpec((BN, D), row),
            pl.BlockSpec((1, BN, D), lambda i: (0, i, 0)),
            pl.BlockSpec((1, BN, D), lambda i: (1, i, 0)),
            pl.BlockSpec(memory_space=pltpu.SMEM),
            pl.BlockSpec((D, D), full),
            pl.BlockSpec((1, D), full),
            pl.BlockSpec((D, D), full),
            pl.BlockSpec((1, D), full),
            pl.BlockSpec((D, D), full),
            pl.BlockSpec((1, D), full),
        ],
        out_specs=pl.BlockSpec((BN, D), row),
        out_shape=jax.ShapeDtypeStruct((N, D), jnp.float32),
    )(feature, agg, agg, eps.reshape(1, 1), W1, b1.reshape(1, D),
      W2, b2.reshape(1, D), Wl, bl.reshape(1, D))


def kernel(feature, adj, edge_attr, We, be, eps, W1, b1, W2, b2, Wl, bl):
    N, D = feature.shape
    E = edge_attr.shape[0]
    adj32 = adj.astype(jnp.int32)

    e = _edge_linear(edge_attr.T, We, be)
    agg = _make_sc_edge(N, E, D)(adj32, feature, e)
    return _mlp(feature, agg, eps, W1, b1, W2, b2, Wl, bl)
